# initial kernel scaffold (unmeasured)
import jax
import jax.numpy as jnp
from jax import lax
from jax.experimental import pallas as pl
from jax.experimental.pallas import tpu as pltpu

N_DEV = 16
B = 2
SQ = 256
SKV = 256
DH = 64
HL = 4
HD = HL * DH
D_MODEL = 512
BLK = 64


def kernel(x, Wq, K_ext, V_ext, Wo):
    K_t = jnp.transpose(K_ext, (2, 0, 1, 3))
    V_t = jnp.transpose(V_ext, (2, 0, 1, 3))

    def body(x_ref, wq_ref, k_ref, v_ref, wo_ref, out_ref,
             comm_wq, comm_wo, s_wq, s_wo, r_wq, r_wo):
        my = lax.axis_index("i")

        sends = []
        for d in range(1, N_DEV):
            tgt = (my + d) % N_DEV
            rd_q = pltpu.make_async_remote_copy(
                src_ref=wq_ref, dst_ref=comm_wq.at[my],
                send_sem=s_wq.at[tgt], recv_sem=r_wq.at[my],
                device_id=(tgt,), device_id_type=pl.DeviceIdType.MESH)
            rd_o = pltpu.make_async_remote_copy(
                src_ref=wo_ref, dst_ref=comm_wo.at[my],
                send_sem=s_wo.at[tgt], recv_sem=r_wo.at[my],
                device_id=(tgt,), device_id_type=pl.DeviceIdType.MESH)
            rd_q.start()
            rd_o.start()
            sends += [rd_q, rd_o]

        ri = lax.broadcasted_iota(jnp.int32, (SQ, SKV), 0)
        ci = lax.broadcasted_iota(jnp.int32, (SQ, SKV), 1)
        qb = my * (SQ // BLK) + ri // BLK
        kb = ci // BLK
        mask = (qb == kb) | (kb == 0) | ((qb + kb) % 3 == 0)

        x2d = x_ref[...].reshape(B * SQ, D_MODEL)

        def accum(o, wq_c, wo_c, first):
            q = jnp.dot(x2d, wq_c, preferred_element_type=jnp.float32)
            kc = k_ref[pl.ds(o * HL, HL)]
            vc = v_ref[pl.ds(o * HL, HL)]
            for b in range(B):
                ctx_cols = []
                for h in range(HL):
                    q_bh = q[b * SQ:(b + 1) * SQ, h * DH:(h + 1) * DH]
                    s = lax.dot_general(
                        q_bh, kc[h, b], (((1,), (1,)), ((), ())),
                        preferred_element_type=jnp.float32) * 0.125
                    s = jnp.where(mask, s, -1e9)
                    m = jnp.max(s, axis=1, keepdims=True)
                    w = jnp.exp(s - m)
                    w = w / jnp.sum(w, axis=1, keepdims=True)
                    ctx_cols.append(jnp.dot(
                        w, vc[h, b], preferred_element_type=jnp.float32))
                ctx = jnp.concatenate(ctx_cols, axis=1)
                part = jnp.dot(ctx, wo_c,
                               preferred_element_type=jnp.float32)
                if first:
                    out_ref[b] = part
                else:
                    out_ref[b] = out_ref[b] + part

        accum(my, wq_ref[...], wo_ref[...], first=True)

        for d in range(1, N_DEV):
            o = (my - d) % N_DEV
            for comm, sems in ((comm_wq, r_wq), (comm_wo, r_wo)):
                recv = pltpu.make_async_remote_copy(
                    src_ref=comm.at[o], dst_ref=comm.at[o],
                    send_sem=sems.at[o], recv_sem=sems.at[o],
                    device_id=(o,), device_id_type=pl.DeviceIdType.MESH)
                recv.wait_recv()
            accum(o, comm_wq[o], comm_wo[o], first=False)

        for rd in sends:
            rd.wait_send()

    out_shape = jax.ShapeDtypeStruct((B, SQ, D_MODEL), jnp.float32)
    return pl.pallas_call(
        body,
        out_shape=out_shape,
        in_specs=[pl.BlockSpec(memory_space=pltpu.VMEM)] * 5,
        out_specs=pl.BlockSpec(memory_space=pltpu.VMEM),
        scratch_shapes=[
            pltpu.VMEM((N_DEV, D_MODEL, HD), jnp.float32),
            pltpu.VMEM((N_DEV, HD, D_MODEL), jnp.float32),
            pltpu.SemaphoreType.DMA((N_DEV,)),
            pltpu.SemaphoreType.DMA((N_DEV,)),
            pltpu.SemaphoreType.DMA((N_DEV,)),
            pltpu.SemaphoreType.DMA((N_DEV,)),
        ],
    )(x, Wq, K_t, V_t, Wo)


# baseline (device time: 58082 ns/iter reference)
import os

import jax
import jax.numpy as jnp
from jax import lax
from jax.experimental import pallas as pl
from jax.experimental.pallas import tpu as pltpu

_PROBE = os.environ.get("KERNEL_PROBE", "")

N_DEV = 16
B = 2
SQ = 256
SKV = 256
DH = 64
HL = 4
HD = HL * DH
D_MODEL = 512
BLK = 64


def kernel(x, Wq, K_ext, V_ext, Wo):
    K_t = jnp.transpose(K_ext, (2, 0, 1, 3)).astype(jnp.bfloat16)
    V_t = jnp.transpose(V_ext, (2, 0, 1, 3)).astype(jnp.bfloat16)
    Wq16 = Wq.astype(jnp.bfloat16)
    Wo16 = Wo.astype(jnp.bfloat16)

    def body(x_ref, wq_ref, k_ref, v_ref, wo_ref, out_ref,
             comm_wq, comm_wo, s_wq, s_wo, r_wq, r_wo):
        my = lax.axis_index("i")

        sends = []
        for d in range(1, N_DEV) if _PROBE != "compute" else ():
            tgt = (my + d) % N_DEV
            rd_q = pltpu.make_async_remote_copy(
                src_ref=wq_ref, dst_ref=comm_wq.at[my],
                send_sem=s_wq.at[tgt], recv_sem=r_wq.at[my],
                device_id=(tgt,), device_id_type=pl.DeviceIdType.MESH)
            rd_o = pltpu.make_async_remote_copy(
                src_ref=wo_ref, dst_ref=comm_wo.at[my],
                send_sem=s_wo.at[tgt], recv_sem=r_wo.at[my],
                device_id=(tgt,), device_id_type=pl.DeviceIdType.MESH)
            rd_q.start()
            rd_o.start()
            sends += [rd_q, rd_o]

        ri = lax.broadcasted_iota(jnp.int32, (SQ, SKV), 0)
        ci = lax.broadcasted_iota(jnp.int32, (SQ, SKV), 1)
        qb = my * (SQ // BLK) + ri // BLK
        kb = ci // BLK
        mask = (qb == kb) | (kb == 0) | ((qb + kb) % 3 == 0)
        bias = jnp.where(mask, 0.0, -1e9).astype(jnp.float32)

        x2d = (x_ref[...].reshape(B * SQ, D_MODEL) * 0.125
               ).astype(jnp.bfloat16)

        def accum(o, wq_c, wo_c, first):
            if _PROBE == "comm":
                part = wq_c[:B * SQ // 2].reshape(B, SQ // 2, HD
                                                  ).astype(jnp.float32)
                if first:
                    out_ref[:, :SQ // 2, :HD] = part
                else:
                    out_ref[:, :SQ // 2, :HD] += part
                return
            q = jnp.dot(x2d, wq_c,
                        preferred_element_type=jnp.float32
                        ).astype(jnp.bfloat16)
            kc = k_ref[pl.ds(o * HL, HL)]
            vc = v_ref[pl.ds(o * HL, HL)]
            for b in range(B):
                ctx_cols = []
                for h in range(HL):
                    q_bh = q[b * SQ:(b + 1) * SQ, h * DH:(h + 1) * DH]
                    s = lax.dot_general(
                        q_bh, kc[h, b], (((1,), (1,)), ((), ())),
                        preferred_element_type=jnp.float32)
                    w = jnp.exp(s + bias)
                    denom = jnp.sum(w, axis=1, keepdims=True)
                    ctx = jnp.dot(w.astype(jnp.bfloat16), vc[h, b],
                                  preferred_element_type=jnp.float32)
                    ctx_cols.append(ctx / denom)
                ctx = jnp.concatenate(ctx_cols, axis=1)
                part = jnp.dot(ctx.astype(jnp.bfloat16), wo_c,
                               preferred_element_type=jnp.float32)
                if first:
                    out_ref[b] = part
                else:
                    out_ref[b] = out_ref[b] + part

        accum(my, wq_ref[...], wo_ref[...], first=True)

        for d in range(1, N_DEV):
            o = (my - d) % N_DEV
            if _PROBE == "compute":
                accum(o, wq_ref[...], wo_ref[...], first=False)
                continue
            for comm, sems in ((comm_wq, r_wq), (comm_wo, r_wo)):
                recv = pltpu.make_async_remote_copy(
                    src_ref=comm.at[o], dst_ref=comm.at[o],
                    send_sem=sems.at[o], recv_sem=sems.at[o],
                    device_id=(o,), device_id_type=pl.DeviceIdType.MESH)
                recv.wait_recv()
            accum(o, comm_wq[o], comm_wo[o], first=False)

        for rd in sends:
            rd.wait_send()

    out_shape = jax.ShapeDtypeStruct((B, SQ, D_MODEL), jnp.float32)
    return pl.pallas_call(
        body,
        out_shape=out_shape,
        in_specs=[pl.BlockSpec(memory_space=pltpu.VMEM)] * 5,
        out_specs=pl.BlockSpec(memory_space=pltpu.VMEM),
        scratch_shapes=[
            pltpu.VMEM((N_DEV, D_MODEL, HD), jnp.bfloat16),
            pltpu.VMEM((N_DEV, HD, D_MODEL), jnp.bfloat16),
            pltpu.SemaphoreType.DMA((N_DEV,)),
            pltpu.SemaphoreType.DMA((N_DEV,)),
            pltpu.SemaphoreType.DMA((N_DEV,)),
            pltpu.SemaphoreType.DMA((N_DEV,)),
        ],
        compiler_params=pltpu.CompilerParams(
            vmem_limit_bytes=64 * 1024 * 1024,
        ),
    )(x, Wq16, K_t, V_t, Wo16)
